# baseline (speedup 1.0000x reference)
import numpy as np

import jax
import jax.numpy as jnp
from jax import lax
from jax.experimental import pallas as pl
from jax.experimental.pallas import tpu as pltpu
from jax.experimental.pallas import tpu_sc as plsc

NUM_FACTORS = 3
VOCAB_P1 = 513
D = 2048
B = 4
S = 8192
N = B * S

NC = 2
NS = 16
LANES = 16
NW = NC * NS
NT = N // NW
T = 8
NCHUNK = NT // T
GROUPS_PER_ROW = D // (2 * LANES)
HIMASK = -65536

_SRC = np.empty((D,), dtype=np.int32)
for _g in range(GROUPS_PER_ROW):
  for _j in range(LANES):
    _SRC[32 * _g + 2 * _j] = 32 * _g + _j
    _SRC[32 * _g + 2 * _j + 1] = 32 * _g + LANES + _j


def _body(w0, w1, w2, i0, i1, i2, out,
          idx0_v, idx1_v, idx2_v,
          ob0, ob1, g1b0, g1b1, g2b0, g2b1,
          s00, s01, s10, s11, s20, s21):
  wid = lax.axis_index("s") * NC + lax.axis_index("c")
  base = wid * NT

  obufs = (ob0, ob1)
  g1bufs = (g1b0, g1b1)
  g2bufs = (g2b0, g2b1)
  sems = ((s00, s10, s20), (s01, s11, s21))

  pltpu.sync_copy(i0.at[wid], idx0_v)
  pltpu.sync_copy(i1.at[wid], idx1_v)
  pltpu.sync_copy(i2.at[wid], idx2_v)

  def issue(c, s):
    pltpu.async_copy(w0.at[idx0_v.at[pl.ds(c * T, T)]], obufs[s], sems[s][0])
    pltpu.async_copy(w1.at[idx1_v.at[pl.ds(c * T, T)]], g1bufs[s], sems[s][1])
    pltpu.async_copy(w2.at[idx2_v.at[pl.ds(c * T, T)]], g2bufs[s], sems[s][2])

  def drain(c, s):
    pltpu.make_async_copy(w0.at[idx0_v.at[pl.ds(c * T, T)]], obufs[s],
                          sems[s][0]).wait()
    pltpu.make_async_copy(w1.at[idx1_v.at[pl.ds(c * T, T)]], g1bufs[s],
                          sems[s][1]).wait()
    pltpu.make_async_copy(w2.at[idx2_v.at[pl.ds(c * T, T)]], g2bufs[s],
                          sems[s][2]).wait()

  def fold_store(c, s):
    ob, g1, g2 = obufs[s], g1bufs[s], g2bufs[s]

    def row_body(r, rcarry):
      def group_body(v, gcarry):
        colw = v * LANES
        col = v * 2 * LANES
        x1 = g1[r, pl.ds(colw, LANES)]
        x2 = g2[r, pl.ds(colw, LANES)]
        a = (lax.bitcast_convert_type(x1 << 16, jnp.float32)
             + lax.bitcast_convert_type(x2 << 16, jnp.float32))
        b = (lax.bitcast_convert_type(x1, jnp.float32)
             + lax.bitcast_convert_type(x2, jnp.float32))
        plsc.addupdate(ob.at[r, pl.ds(col, LANES)], a)
        plsc.addupdate(ob.at[r, pl.ds(col + LANES, LANES)], b)
        return gcarry

      lax.fori_loop(0, GROUPS_PER_ROW, group_body, 0, unroll=16)
      return rcarry

    lax.fori_loop(0, T, row_body, 0, unroll=False)
    pltpu.sync_copy(ob, out.at[pl.ds(base + c * T, T)])

  issue(0, 0)

  def pair_body(p, carry):
    c0 = 2 * p
    c1 = c0 + 1
    c2 = jnp.minimum(c0 + 2, NCHUNK - 1)
    issue(c1, 1)
    drain(c0, 0)
    fold_store(c0, 0)
    issue(c2, 0)
    drain(c1, 1)
    fold_store(c1, 1)
    return carry

  lax.fori_loop(0, NCHUNK // 2, pair_body, 0, unroll=False)
  drain(NCHUNK - 1, 0)


@jax.jit
def kernel(x, W0, W1, W2):
  src = jnp.asarray(_SRC)

  def prep(w):
    wb = w[:, src].astype(jnp.bfloat16).reshape(VOCAB_P1, D // 2, 2)
    return lax.bitcast_convert_type(wb, jnp.int32)

  wb1, wb2 = prep(W1), prep(W2)
  xt = jnp.transpose(x.astype(jnp.int32), (1, 0, 2)).reshape(
      NUM_FACTORS, NW, NT)
  mesh = plsc.VectorSubcoreMesh(core_axis_name="c", subcore_axis_name="s",
                                num_cores=NC, num_subcores=NS)
  fn = pl.kernel(
      _body,
      out_type=jax.ShapeDtypeStruct((N, D), jnp.float32),
      mesh=mesh,
      scratch_types=[
          pltpu.VMEM((NT,), jnp.int32),
          pltpu.VMEM((NT,), jnp.int32),
          pltpu.VMEM((NT,), jnp.int32),
          pltpu.VMEM((T, D), jnp.float32),
          pltpu.VMEM((T, D), jnp.float32),
          pltpu.VMEM((T, D // 2), jnp.int32),
          pltpu.VMEM((T, D // 2), jnp.int32),
          pltpu.VMEM((T, D // 2), jnp.int32),
          pltpu.VMEM((T, D // 2), jnp.int32),
          pltpu.SemaphoreType.DMA,
          pltpu.SemaphoreType.DMA,
          pltpu.SemaphoreType.DMA,
          pltpu.SemaphoreType.DMA,
          pltpu.SemaphoreType.DMA,
          pltpu.SemaphoreType.DMA,
      ],
  )
  out = fn(W0, wb1, wb2, xt[0], xt[1], xt[2])
  return out.reshape(B, S, D)


# speedup vs baseline: 1.1952x; 1.1952x over previous
import numpy as np

import jax
import jax.numpy as jnp
from jax import lax
from jax.experimental import pallas as pl
from jax.experimental.pallas import tpu as pltpu
from jax.experimental.pallas import tpu_sc as plsc

NUM_FACTORS = 3
VOCAB_P1 = 513
D = 2048
B = 4
S = 8192
N = B * S

NC = 2
NS = 16
LANES = 16
NW = NC * NS
NT = N // NW
T = 8
NCHUNK = NT // T
GROUPS_PER_ROW = D // (2 * LANES)
HIMASK = -65536

_SRC = np.empty((D,), dtype=np.int32)
for _g in range(GROUPS_PER_ROW):
  for _j in range(LANES):
    _SRC[32 * _g + 2 * _j] = 32 * _g + _j
    _SRC[32 * _g + 2 * _j + 1] = 32 * _g + LANES + _j


def _body(w0, w1, w2, i0, i1, i2, out,
          idx0_v, idx1_v, idx2_v,
          ob0, ob1, g1b0, g1b1, g2b0, g2b1,
          s00, s01, s10, s11, s20, s21, st0, st1):
  wid = lax.axis_index("s") * NC + lax.axis_index("c")
  base = wid * NT

  obufs = (ob0, ob1)
  g1bufs = (g1b0, g1b1)
  g2bufs = (g2b0, g2b1)
  sems = ((s00, s10, s20), (s01, s11, s21))
  stsems = (st0, st1)

  pltpu.sync_copy(i0.at[wid], idx0_v)
  pltpu.sync_copy(i1.at[wid], idx1_v)
  pltpu.sync_copy(i2.at[wid], idx2_v)

  def issue(c, s):
    pltpu.async_copy(w0.at[idx0_v.at[pl.ds(c * T, T)]], obufs[s], sems[s][0])
    pltpu.async_copy(w1.at[idx1_v.at[pl.ds(c * T, T)]], g1bufs[s], sems[s][1])
    pltpu.async_copy(w2.at[idx2_v.at[pl.ds(c * T, T)]], g2bufs[s], sems[s][2])

  def drain(c, s):
    pltpu.make_async_copy(w0.at[idx0_v.at[pl.ds(c * T, T)]], obufs[s],
                          sems[s][0]).wait()
    pltpu.make_async_copy(w1.at[idx1_v.at[pl.ds(c * T, T)]], g1bufs[s],
                          sems[s][1]).wait()
    pltpu.make_async_copy(w2.at[idx2_v.at[pl.ds(c * T, T)]], g2bufs[s],
                          sems[s][2]).wait()

  def fold_store(c, s):
    ob, g1, g2 = obufs[s], g1bufs[s], g2bufs[s]

    def row_body(r, rcarry):
      for v in range(GROUPS_PER_ROW):
        colw = v * LANES
        col = v * 2 * LANES
        x1 = g1[r, pl.ds(colw, LANES)]
        x2 = g2[r, pl.ds(colw, LANES)]
        a = (lax.bitcast_convert_type(x1 << 16, jnp.float32)
             + lax.bitcast_convert_type(x2 << 16, jnp.float32))
        b = (lax.bitcast_convert_type(x1, jnp.float32)
             + lax.bitcast_convert_type(x2, jnp.float32))
        plsc.addupdate(ob.at[r, pl.ds(col, LANES)], a)
        plsc.addupdate(ob.at[r, pl.ds(col + LANES, LANES)], b)
      return rcarry

    lax.fori_loop(0, T, row_body, 0, unroll=False)
    pltpu.async_copy(ob, out.at[pl.ds(base + c * T, T)], stsems[s])

  def drain_store(s):
    pltpu.make_async_copy(obufs[s], out.at[pl.ds(base, T)], stsems[s]).wait()

  issue(0, 0)

  def pair_body(p, carry):
    c0 = 2 * p
    c1 = c0 + 1
    c2 = jnp.minimum(c0 + 2, NCHUNK - 1)

    @pl.when(p > 0)
    def _():
      drain_store(1)

    issue(c1, 1)
    drain(c0, 0)
    fold_store(c0, 0)
    drain(c1, 1)
    drain_store(0)
    issue(c2, 0)
    fold_store(c1, 1)
    return carry

  lax.fori_loop(0, NCHUNK // 2, pair_body, 0, unroll=False)
  drain_store(1)
  drain(NCHUNK - 1, 0)


@jax.jit
def kernel(x, W0, W1, W2):
  src = jnp.asarray(_SRC)

  def prep(w):
    wb = w[:, src].astype(jnp.bfloat16).reshape(VOCAB_P1, D // 2, 2)
    return lax.bitcast_convert_type(wb, jnp.int32)

  wb1, wb2 = prep(W1), prep(W2)
  xt = jnp.transpose(x.astype(jnp.int32), (1, 0, 2)).reshape(
      NUM_FACTORS, NW, NT)
  mesh = plsc.VectorSubcoreMesh(core_axis_name="c", subcore_axis_name="s",
                                num_cores=NC, num_subcores=NS)
  fn = pl.kernel(
      _body,
      out_type=jax.ShapeDtypeStruct((N, D), jnp.float32),
      mesh=mesh,
      scratch_types=[
          pltpu.VMEM((NT,), jnp.int32),
          pltpu.VMEM((NT,), jnp.int32),
          pltpu.VMEM((NT,), jnp.int32),
          pltpu.VMEM((T, D), jnp.float32),
          pltpu.VMEM((T, D), jnp.float32),
          pltpu.VMEM((T, D // 2), jnp.int32),
          pltpu.VMEM((T, D // 2), jnp.int32),
          pltpu.VMEM((T, D // 2), jnp.int32),
          pltpu.VMEM((T, D // 2), jnp.int32),
          pltpu.SemaphoreType.DMA,
          pltpu.SemaphoreType.DMA,
          pltpu.SemaphoreType.DMA,
          pltpu.SemaphoreType.DMA,
          pltpu.SemaphoreType.DMA,
          pltpu.SemaphoreType.DMA,
          pltpu.SemaphoreType.DMA,
          pltpu.SemaphoreType.DMA,
      ],
  )
  out = fn(W0, wb1, wb2, xt[0], xt[1], xt[2])
  return out.reshape(B, S, D)
